# baseline (speedup 1.0000x reference)
import functools

import jax
import jax.numpy as jnp
from jax import lax
from jax.experimental import pallas as pl
from jax.experimental.pallas import tpu as pltpu
from jax.experimental.pallas import tpu_sc as plsc

_NC = 2
_NS = 16
_NW = _NC * _NS

_BATCH = 4
_SEQ = 8192
_B = _BATCH * _SEQ
_D = 1024
_BPW = _B // _NW
_WPB = _SEQ // _BPW

_C = 8
_NB = 8
_L = _NB // 2
_NCHUNK = _BPW // _C


def _make_sc_gather():
    mesh = plsc.VectorSubcoreMesh(core_axis_name="c", subcore_axis_name="s")

    @functools.partial(
        pl.kernel,
        mesh=mesh,
        out_type=jax.ShapeDtypeStruct((_B, _D), jnp.float32),
        scratch_types=(
            [pltpu.VMEM((_BPW,), jnp.int32)]
            + [pltpu.VMEM((_C, _D), jnp.float32) for _ in range(_NB)]
            + [pltpu.SemaphoreType.DMA, pltpu.SemaphoreType.DMA]
        ),
    )
    def gather_kernel(table_hbm, idx_hbm, out_hbm, idx_v, *rest):
        bufs = rest[:_NB]
        gsem, wsem = rest[_NB], rest[_NB + 1]
        wid = lax.axis_index("s") * _NC + lax.axis_index("c")
        base = wid * _BPW
        pltpu.sync_copy(
            idx_hbm.at[wid // _WPB, pl.ds((wid % _WPB) * _BPW, _BPW)], idx_v
        )
        gathers = [None] * _NB
        writes = [None] * _NB

        def start_gather(c):
            return pltpu.async_copy(
                table_hbm.at[idx_v.at[pl.ds(c * _C, _C)]], bufs[c % _NB], gsem
            )

        for j in range(_L):
            gathers[j] = start_gather(j)
        for c in range(_NCHUNK):
            if c >= _L:
                writes[(c - _L) % _NB].wait()
            if c + _L < _NCHUNK:
                gathers[(c + _L) % _NB] = start_gather(c + _L)
            gathers[c % _NB].wait()
            writes[c % _NB] = pltpu.async_copy(
                bufs[c % _NB], out_hbm.at[pl.ds(base + c * _C, _C)], wsem
            )
        for j in range(_L):
            writes[(_NCHUNK - _L + j) % _NB].wait()

    return gather_kernel


_sc_gather = _make_sc_gather()


def kernel(x, table):
    out = _sc_gather(table, x)
    return out.reshape(x.shape + (table.shape[1],))


# speedup vs baseline: 1.0173x; 1.0173x over previous
import functools

import jax
import jax.numpy as jnp
from jax import lax
from jax.experimental import pallas as pl
from jax.experimental.pallas import tpu as pltpu
from jax.experimental.pallas import tpu_sc as plsc

_NC = 2
_NS = 16
_NW = _NC * _NS

_BATCH = 4
_SEQ = 8192
_B = _BATCH * _SEQ
_D = 1024
_BPW = _B // _NW
_WPB = _SEQ // _BPW

_C = 16
_NB = 4
_L = _NB // 2
_NCHUNK = _BPW // _C


def _make_sc_gather():
    mesh = plsc.VectorSubcoreMesh(core_axis_name="c", subcore_axis_name="s")

    @functools.partial(
        pl.kernel,
        mesh=mesh,
        out_type=jax.ShapeDtypeStruct((_B, _D), jnp.float32),
        scratch_types=(
            [pltpu.VMEM((_BPW,), jnp.int32)]
            + [pltpu.VMEM((_C, _D), jnp.float32) for _ in range(_NB)]
            + [pltpu.SemaphoreType.DMA, pltpu.SemaphoreType.DMA]
        ),
    )
    def gather_kernel(table_hbm, idx_hbm, out_hbm, idx_v, *rest):
        bufs = rest[:_NB]
        gsem, wsem = rest[_NB], rest[_NB + 1]
        wid = lax.axis_index("s") * _NC + lax.axis_index("c")
        base = wid * _BPW
        pltpu.sync_copy(
            idx_hbm.at[wid // _WPB, pl.ds((wid % _WPB) * _BPW, _BPW)], idx_v
        )
        gathers = [None] * _NB
        writes = [None] * _NB

        def start_gather(c):
            return pltpu.async_copy(
                table_hbm.at[idx_v.at[pl.ds(c * _C, _C)]], bufs[c % _NB], gsem
            )

        for j in range(_L):
            gathers[j] = start_gather(j)
        for c in range(_NCHUNK):
            if c >= _L:
                writes[(c - _L) % _NB].wait()
            if c + _L < _NCHUNK:
                gathers[(c + _L) % _NB] = start_gather(c + _L)
            gathers[c % _NB].wait()
            writes[c % _NB] = pltpu.async_copy(
                bufs[c % _NB], out_hbm.at[pl.ds(base + c * _C, _C)], wsem
            )
        for j in range(_L):
            writes[(_NCHUNK - _L + j) % _NB].wait()

    return gather_kernel


_sc_gather = _make_sc_gather()


def kernel(x, table):
    out = _sc_gather(table, x)
    return out.reshape(x.shape + (table.shape[1],))
